# Initial kernel scaffold; baseline (speedup 1.0000x reference)
import jax
import jax.numpy as jnp
from jax.experimental import pallas as pl


def kernel(X, S, Wg, Ws, msg_W1, msg_b1, msg_W2, msg_b2, gate_W1, gate_b1, gate_W2, gate_b2, attn_in_W, attn_in_b, attn_out_W, attn_out_b, attn_ln_g, attn_ln_b, upd_W1, upd_b1, upd_W2, upd_b2, ln_g, ln_b):
    raise NotImplementedError("write your pallas kernel here")


# speedup vs baseline: 3.2323x; 3.2323x over previous
import functools

import jax
import jax.numpy as jnp
from jax.experimental import pallas as pl
from jax.experimental.pallas import tpu as pltpu

B, L, D, G, K, N, H = 4, 4096, 1024, 16, 8, 128, 16
DH = D // H
T = 512


def _gelu(x):
    return x * 0.5 * (1.0 + jax.lax.erf(x * (2.0 ** -0.5)))


def _ln(x, g, b, eps=1e-5):
    m = jnp.mean(x, axis=-1, keepdims=True)
    v = jnp.mean((x - m) ** 2, axis=-1, keepdims=True)
    return (x - m) * jax.lax.rsqrt(v + eps) * g + b


def _token_kernel(x_ref, wcat_ref, bcat_ref, gw2_ref, gb2_ref, wscore_ref,
                  acc_ref):
    t = pl.program_id(1)

    x = x_ref[0]
    u = jax.lax.dot_general(x, wcat_ref[...], (((1,), (1,)), ((), ())),
                            preferred_element_type=jnp.float32)
    u = u + bcat_ref[...]
    h = _gelu(u)
    h_msg = h[:, :2 * D]
    h_gate = h[:, 2 * D:]

    gate_logit = jax.lax.dot_general(h_gate, gw2_ref[...],
                                     (((1,), (1,)), ((), ())),
                                     preferred_element_type=jnp.float32)
    gate = jax.nn.sigmoid(gate_logit[:, :1] + gb2_ref[0, 0])

    sc = jax.lax.dot_general(x, wscore_ref[...], (((1,), (1,)), ((), ())),
                             preferred_element_type=jnp.float32)
    ag = jnp.argmax(sc[:, :G], axis=-1, keepdims=True)
    ak = jnp.argmax(sc[:, G:G + K], axis=-1, keepdims=True)
    n_idx = ag * K + ak
    lanes = jax.lax.broadcasted_iota(jnp.int32, (T, N), 1)
    onehot = (lanes == n_idx).astype(jnp.float32)

    gh = h_msg * gate
    gate_pad = gate * (lanes == 0).astype(jnp.float32)
    rhs = jnp.concatenate([gh, gate_pad], axis=1)
    part = jax.lax.dot_general(onehot, rhs, (((0,), (0,)), ((), ())),
                               preferred_element_type=jnp.float32)

    @pl.when(t == 0)
    def _init():
        acc_ref[0] = part

    @pl.when(t != 0)
    def _acc():
        acc_ref[0] += part


def _slot_kernel(s_ref, a_ref, gsum_ref,
                 inw_ref, inb_ref, outw_ref, outb_ref, alng_ref, alnb_ref,
                 mw2_ref, mb2_ref, uw1_ref, ub1_ref, uw2_ref, ub2_ref,
                 lng_ref, lnb_ref, o_ref):
    s = s_ref[0]
    qkv = jax.lax.dot_general(s, inw_ref[...], (((1,), (1,)), ((), ())),
                              preferred_element_type=jnp.float32)
    qkv = qkv + inb_ref[...]
    q = qkv[:, :D]
    k = qkv[:, D:2 * D]
    v = qkv[:, 2 * D:]

    scale = 1.0 / (DH ** 0.5)
    outs = []
    for hh in range(H):
        sl = slice(hh * DH, (hh + 1) * DH)
        qh, kh, vh = q[:, sl], k[:, sl], v[:, sl]
        sc = jax.lax.dot_general(qh, kh, (((1,), (1,)), ((), ())),
                                 preferred_element_type=jnp.float32) * scale
        m = jnp.max(sc, axis=-1, keepdims=True)
        e = jnp.exp(sc - m)
        a = e / jnp.sum(e, axis=-1, keepdims=True)
        outs.append(jax.lax.dot_general(a, vh, (((1,), (0,)), ((), ())),
                                        preferred_element_type=jnp.float32))
    o = jnp.concatenate(outs, axis=1)

    attn_out = jax.lax.dot_general(o, outw_ref[...], (((1,), (1,)), ((), ())),
                                   preferred_element_type=jnp.float32)
    attn_out = attn_out + outb_ref[...]
    s1 = _ln(s + attn_out, alng_ref[...], alnb_ref[...])

    incoming = jax.lax.dot_general(a_ref[0], mw2_ref[...],
                                   (((1,), (1,)), ((), ())),
                                   preferred_element_type=jnp.float32)
    incoming = incoming + gsum_ref[0] * mb2_ref[...]

    cat = jnp.concatenate([s1, incoming], axis=1)
    hid = _gelu(jax.lax.dot_general(cat, uw1_ref[...], (((1,), (1,)), ((), ())),
                                    preferred_element_type=jnp.float32)
                + ub1_ref[...])
    upd = jax.lax.dot_general(hid, uw2_ref[...], (((1,), (1,)), ((), ())),
                              preferred_element_type=jnp.float32)
    upd = upd + ub2_ref[...]
    o_ref[0] = _ln(s1 + upd, lng_ref[...], lnb_ref[...])


def kernel(X, S, Wg, Ws, msg_W1, msg_b1, msg_W2, msg_b2, gate_W1, gate_b1,
           gate_W2, gate_b2, attn_in_W, attn_in_b, attn_out_W, attn_out_b,
           attn_ln_g, attn_ln_b, upd_W1, upd_b1, upd_W2, upd_b2, ln_g, ln_b):
    wcat = jnp.concatenate([msg_W1, gate_W1], axis=0)
    bcat = jnp.concatenate([msg_b1, gate_b1]).reshape(1, 3 * D)
    wscore = jnp.concatenate([Wg, Ws], axis=0)

    row = lambda a: a.reshape(1, -1)

    call = pl.pallas_call(
        _token_kernel,
        grid=(B, L // T),
        in_specs=[
            pl.BlockSpec((1, T, D), lambda b, t: (b, t, 0)),
            pl.BlockSpec((3 * D, D), lambda b, t: (0, 0)),
            pl.BlockSpec((1, 3 * D), lambda b, t: (0, 0)),
            pl.BlockSpec((N, D), lambda b, t: (0, 0)),
            pl.BlockSpec(memory_space=pltpu.SMEM),
            pl.BlockSpec((G + K, D), lambda b, t: (0, 0)),
        ],
        out_specs=pl.BlockSpec((1, N, 2 * D + N), lambda b, t: (b, 0, 0)),
        out_shape=jax.ShapeDtypeStruct((B, N, 2 * D + N), jnp.float32),
    )
    gw2_pad = jnp.zeros((N, D), jnp.float32).at[0].set(gate_W2[0])
    accfull = call(X, wcat, bcat, gw2_pad, gate_b2.reshape(1, 1), wscore)

    acc = accfull[:, :, :2 * D]
    gsum_t = accfull[:, :, 2 * D:2 * D + 1]

    out = pl.pallas_call(
        _slot_kernel,
        grid=(B,),
        in_specs=[
            pl.BlockSpec((1, N, D), lambda b: (b, 0, 0)),
            pl.BlockSpec((1, N, 2 * D), lambda b: (b, 0, 0)),
            pl.BlockSpec((1, N, 1), lambda b: (b, 0, 0)),
            pl.BlockSpec((3 * D, D), lambda b: (0, 0)),
            pl.BlockSpec((1, 3 * D), lambda b: (0, 0)),
            pl.BlockSpec((D, D), lambda b: (0, 0)),
            pl.BlockSpec((1, D), lambda b: (0, 0)),
            pl.BlockSpec((1, D), lambda b: (0, 0)),
            pl.BlockSpec((1, D), lambda b: (0, 0)),
            pl.BlockSpec((D, 2 * D), lambda b: (0, 0)),
            pl.BlockSpec((1, D), lambda b: (0, 0)),
            pl.BlockSpec((2 * D, 2 * D), lambda b: (0, 0)),
            pl.BlockSpec((1, 2 * D), lambda b: (0, 0)),
            pl.BlockSpec((D, 2 * D), lambda b: (0, 0)),
            pl.BlockSpec((1, D), lambda b: (0, 0)),
            pl.BlockSpec((1, D), lambda b: (0, 0)),
            pl.BlockSpec((1, D), lambda b: (0, 0)),
        ],
        out_specs=pl.BlockSpec((1, N, D), lambda b: (b, 0, 0)),
        out_shape=jax.ShapeDtypeStruct((B, N, D), jnp.float32),
    )(S, acc, gsum_t,
      attn_in_W, row(attn_in_b), attn_out_W, row(attn_out_b),
      row(attn_ln_g), row(attn_ln_b), msg_W2, row(msg_b2),
      upd_W1, row(upd_b1), upd_W2, row(upd_b2), row(ln_g), row(ln_b))

    return out
